# baseline (speedup 1.0000x reference)
import jax
import jax.numpy as jnp
from jax.experimental import pallas as pl

NUM_SPECIES = 4
B, A, AEV = 1024, 32, 384
MOL_BLK = 16
TOK_BLK = MOL_BLK * A


def _celu(x):
    return jnp.where(x > 0, x, jnp.exp(jnp.minimum(x, 0.0)) - 1.0)


def _fused_kernel(s_ref, x_ref, w1_ref, b1_ref, w2_ref, b2_ref,
                  w3_ref, b3_ref, w4_ref, b4_ref, o_ref):
    x = x_ref[...]
    s = s_ref[...]
    acc = jnp.zeros((TOK_BLK, 1), dtype=jnp.float32)
    for i in range(NUM_SPECIES):
        h = _celu(jnp.dot(x, w1_ref[i], preferred_element_type=jnp.float32)
                  + b1_ref[i])
        h = _celu(jnp.dot(h, w2_ref[i], preferred_element_type=jnp.float32)
                  + b2_ref[i])
        h = _celu(jnp.dot(h, w3_ref[i], preferred_element_type=jnp.float32)
                  + b3_ref[i])
        e = (jnp.dot(h, w4_ref[i], preferred_element_type=jnp.float32)
             + b4_ref[i])
        acc = acc + jnp.where(s == i, e, 0.0)
    row = jax.lax.broadcasted_iota(jnp.int32, (MOL_BLK, TOK_BLK), 0)
    col = jax.lax.broadcasted_iota(jnp.int32, (MOL_BLK, TOK_BLK), 1)
    sel = (col // A == row).astype(jnp.float32)
    o_ref[0] = jnp.dot(sel, acc, preferred_element_type=jnp.float32)


def kernel(species, aev, W1, b1, W2, b2, W3, b3, W4, b4):
    s_flat = species.reshape(B * A, 1)
    x_flat = aev.reshape(B * A, AEV)
    nblk = B // MOL_BLK
    full = lambda arr: pl.BlockSpec(arr.shape, lambda b: (0,) * arr.ndim)
    out = pl.pallas_call(
        _fused_kernel,
        grid=(nblk,),
        in_specs=[
            pl.BlockSpec((TOK_BLK, 1), lambda b: (b, 0)),
            pl.BlockSpec((TOK_BLK, AEV), lambda b: (b, 0)),
            full(W1), full(b1), full(W2), full(b2),
            full(W3), full(b3), full(W4), full(b4),
        ],
        out_specs=pl.BlockSpec((1, MOL_BLK, 1), lambda b: (b, 0, 0)),
        out_shape=jax.ShapeDtypeStruct((nblk, MOL_BLK, 1), jnp.float32),
    )(s_flat, x_flat, W1, b1, W2, b2, W3, b3, W4, b4)
    return out.reshape(B)
